# baseline (speedup 1.0000x reference)
import functools

import jax
import jax.numpy as jnp
from jax import lax
from jax.experimental import pallas as pl
from jax.experimental.pallas import tpu as pltpu
from jax.experimental.pallas import tpu_sc as plsc

_L = 16
_NC = 2
_NS = 16
_NW = _NC * _NS
_FP = 32
_IG = 128


def _bcast_lane(vec, lane):
    idx = jnp.full((_L, 1), lane, dtype=jnp.int32)
    dn = lax.GatherDimensionNumbers(
        offset_dims=(), collapsed_slice_dims=(0,), start_index_map=(0,))
    return lax.gather(vec, idx, dn, (1,),
                      mode=lax.GatherScatterMode.PROMISE_IN_BOUNDS)


@functools.lru_cache(maxsize=None)
def _make_fm(B, F, C):
    BPW = B // _NW
    NCHUNK = BPW // C
    G = C * _FP // _IG

    mesh = plsc.VectorSubcoreMesh(core_axis_name="c", subcore_axis_name="s")

    @functools.partial(
        pl.kernel,
        out_type=jax.ShapeDtypeStruct((B * _L,), jnp.float32),
        mesh=mesh,
        compiler_params=pltpu.CompilerParams(use_tc_tiling_on_sc=False),
        scratch_types=[
            pltpu.VMEM((G, _IG), jnp.int32),
            pltpu.VMEM((C, _FP), jnp.float32),
            pltpu.VMEM((C * _FP, _L), jnp.float32),
            pltpu.VMEM((C * _FP,), jnp.float32),
            pltpu.VMEM((C * _L,), jnp.float32),
            pltpu.SemaphoreType.DMA,
        ],
    )
    def fm(idx_hbm, vals_hbm, emb_hbm, w_hbm, out_hbm,
           idx_v, vals_v, rows_v, w_v, tot_v, sem):
        wid = lax.axis_index("s") * _NC + lax.axis_index("c")
        base = wid * BPW

        def chunk_body(ci, carry):
            cbase = pl.multiple_of(base + ci * C, C)
            gbase = pl.multiple_of(cbase * _FP // _IG, _FP * C // _IG)
            pltpu.sync_copy(idx_hbm.at[pl.ds(gbase, G)], idx_v)
            pltpu.sync_copy(vals_hbm.at[pl.ds(cbase, C)], vals_v)
            copies = []
            for g in range(G):
                copies.append(pltpu.async_copy(
                    emb_hbm.at[idx_v.at[g]],
                    rows_v.at[pl.ds(g * _IG, _IG)], sem))
                copies.append(pltpu.async_copy(
                    w_hbm.at[idx_v.at[g]],
                    w_v.at[pl.ds(g * _IG, _IG)], sem))
            for cp in copies:
                cp.wait()

            def row_body(b, rcarry):
                va = vals_v[b, 0:_L]
                vb = vals_v[b, _L:_FP]
                acc_s = jnp.zeros((_L,), jnp.float32)
                acc_q = jnp.zeros((_L,), jnp.float32)
                for f in range(F):
                    row = rows_v[b * _FP + f, :]
                    valv = _bcast_lane(va if f < _L else vb, f % _L)
                    t = row * valv
                    acc_s = acc_s + t
                    acc_q = acc_q + t * t
                wa = w_v[pl.ds(b * _FP, _L)]
                wb = w_v[pl.ds(b * _FP + _L, _L)]
                tot_v[pl.ds(b * _L, _L)] = (0.5 * (acc_s * acc_s - acc_q)
                                            + va * wa + vb * wb)
                return rcarry

            lax.fori_loop(0, C, row_body, 0)
            pltpu.sync_copy(tot_v, out_hbm.at[pl.ds(cbase * _L, C * _L)])
            return carry

        lax.fori_loop(0, NCHUNK, chunk_body, 0)

    return fm


def _tc_finish(t_ref, bias_ref, o_ref):
    x = jnp.sum(t_ref[...], axis=1, keepdims=True) + bias_ref[0]
    o_ref[...] = 1.0 / (1.0 + jnp.exp(-x))


@functools.lru_cache(maxsize=None)
def _make_finish(B):
    BLK = 2048
    return pl.pallas_call(
        _tc_finish,
        grid=(B // BLK,),
        in_specs=[
            pl.BlockSpec((BLK, _L), lambda i: (i, 0)),
            pl.BlockSpec(memory_space=pltpu.SMEM),
        ],
        out_specs=pl.BlockSpec((BLK, 1), lambda i: (i, 0)),
        out_shape=jax.ShapeDtypeStruct((B, 1), jnp.float32),
    )


@jax.jit
def kernel(feature_idx, feature_vals, feature_embedding, linear_w, bias):
    B, F = feature_idx.shape
    idx_p = jnp.pad(feature_idx, ((0, 0), (0, _FP - F)))
    idx_p = idx_p.reshape(B * _FP // _IG, _IG)
    vals_p = jnp.pad(feature_vals, ((0, 0), (0, _FP - F)))
    w_flat = linear_w.reshape(-1)
    tots = _make_fm(B, F, 128)(idx_p, vals_p, feature_embedding, w_flat)
    return _make_finish(B)(tots.reshape(B, _L), bias)


# speedup vs baseline: 1.8560x; 1.8560x over previous
import functools

import jax
import jax.numpy as jnp
from jax import lax
from jax.experimental import pallas as pl
from jax.experimental.pallas import tpu as pltpu
from jax.experimental.pallas import tpu_sc as plsc

_L = 16
_NC = 2
_NS = 16
_NW = _NC * _NS


def _bcast_lane(vec, lane):
    idx = jnp.full((_L, 1), lane, dtype=jnp.int32)
    dn = lax.GatherDimensionNumbers(
        offset_dims=(), collapsed_slice_dims=(0,), start_index_map=(0,))
    return lax.gather(vec, idx, dn, (1,),
                      mode=lax.GatherScatterMode.PROMISE_IN_BOUNDS)


@functools.lru_cache(maxsize=None)
def _make_fm(B, F, C):
    BPW = B // _NW
    NCHUNK = BPW // C
    N = C * F

    mesh = plsc.VectorSubcoreMesh(core_axis_name="c", subcore_axis_name="s")

    @functools.partial(
        pl.kernel,
        out_type=jax.ShapeDtypeStruct((B * _L,), jnp.float32),
        mesh=mesh,
        compiler_params=pltpu.CompilerParams(use_tc_tiling_on_sc=False,
                                             needs_layout_passes=False),
        scratch_types=[
            pltpu.VMEM((N,), jnp.int32),
            pltpu.VMEM((N,), jnp.float32),
            pltpu.VMEM((N, _L), jnp.float32),
            pltpu.VMEM((N,), jnp.float32),
            pltpu.VMEM((C * _L,), jnp.float32),
            pltpu.SemaphoreType.DMA,
        ],
    )
    def fm(idx_hbm, vals_hbm, emb_hbm, w_hbm, out_hbm,
           idx_v, vals_v, rows_v, w_v, tot_v, sem):
        wid = lax.axis_index("s") * _NC + lax.axis_index("c")
        base = wid * BPW
        lanes = lax.iota(jnp.int32, _L)
        lin_mask = (lanes >= (2 * _L - F)).astype(jnp.float32)

        def chunk_body(ci, carry):
            cbase = pl.multiple_of(base + ci * C, C)
            fbase = pl.multiple_of(cbase * F, C * F)
            pltpu.sync_copy(idx_hbm.at[pl.ds(fbase, N)], idx_v)
            pltpu.sync_copy(vals_hbm.at[pl.ds(fbase, N)], vals_v)
            cp_e = pltpu.async_copy(emb_hbm.at[idx_v], rows_v, sem)
            cp_w = pltpu.async_copy(w_hbm.at[idx_v], w_v, sem)
            cp_e.wait()
            cp_w.wait()

            def row_body(b, rcarry):
                va = vals_v[pl.ds(b * F, _L)]
                vb = vals_v[pl.ds(b * F + F - _L, _L)]
                acc_s = jnp.zeros((_L,), jnp.float32)
                acc_q = jnp.zeros((_L,), jnp.float32)
                for f in range(F):
                    row = rows_v[b * F + f, :]
                    if f < _L:
                        valv = _bcast_lane(va, f)
                    else:
                        valv = _bcast_lane(vb, f - (F - _L))
                    t = row * valv
                    acc_s = acc_s + t
                    acc_q = acc_q + t * t
                wa = w_v[pl.ds(b * F, _L)]
                wb = w_v[pl.ds(b * F + F - _L, _L)]
                tot_v[pl.ds(b * _L, _L)] = (0.5 * (acc_s * acc_s - acc_q)
                                            + va * wa + lin_mask * (vb * wb))
                return rcarry

            lax.fori_loop(0, C, row_body, 0)
            pltpu.sync_copy(tot_v, out_hbm.at[pl.ds(cbase * _L, C * _L)])
            return carry

        lax.fori_loop(0, NCHUNK, chunk_body, 0)

    return fm


def _tc_finish(t_ref, bias_ref, o_ref):
    x = jnp.sum(t_ref[...], axis=1, keepdims=True) + bias_ref[0]
    o_ref[...] = 1.0 / (1.0 + jnp.exp(-x))


@functools.lru_cache(maxsize=None)
def _make_finish(B):
    BLK = 2048
    return pl.pallas_call(
        _tc_finish,
        grid=(B // BLK,),
        in_specs=[
            pl.BlockSpec((BLK, _L), lambda i: (i, 0)),
            pl.BlockSpec(memory_space=pltpu.SMEM),
        ],
        out_specs=pl.BlockSpec((BLK, 1), lambda i: (i, 0)),
        out_shape=jax.ShapeDtypeStruct((B, 1), jnp.float32),
    )


@jax.jit
def kernel(feature_idx, feature_vals, feature_embedding, linear_w, bias):
    B, F = feature_idx.shape
    idx_flat = feature_idx.reshape(-1)
    vals_flat = feature_vals.reshape(-1)
    w_flat = linear_w.reshape(-1)
    tots = _make_fm(B, F, 128)(idx_flat, vals_flat, feature_embedding, w_flat)
    return _make_finish(B)(tots.reshape(B, _L), bias)
